# baseline (speedup 1.0000x reference)
import jax
import jax.numpy as jnp
from jax.experimental import pallas as pl
from jax.experimental.pallas import tpu as pltpu

_CH = 1024
_NBUF = 4


def _schedule(rows):
    ramp = [64, 64, 128, 256]
    tail = [256, 128, 64, 64]
    body = rows - sum(ramp) - sum(tail)
    sizes = ramp + [_CH] * (body // _CH) + tail
    chunks, off = [], 0
    for sz in sizes:
        chunks.append((off, sz))
        off += sz
    assert off == rows
    return chunks


def _copy_body(x_ref, o_ref, buf, ld_sems, st_sems):
    chunks = _schedule(x_ref.shape[0])
    n = len(chunks)

    def ld(g):
        slot = g % _NBUF
        off, sz = chunks[g]
        return pltpu.make_async_copy(
            x_ref.at[pl.ds(off, sz)], buf.at[slot, pl.ds(0, sz)],
            ld_sems.at[slot])

    def st(g):
        slot = g % _NBUF
        off, sz = chunks[g]
        return pltpu.make_async_copy(
            buf.at[slot, pl.ds(0, sz)], o_ref.at[pl.ds(off, sz)],
            st_sems.at[slot])

    k = _NBUF // 2
    for g in range(min(k, n)):
        ld(g).start()
    for g in range(n):
        ld(g).wait()
        st(g).start()
        nxt = g + k
        if nxt < n:
            prev = nxt - _NBUF
            if prev >= 0:
                st(prev).wait()
            ld(nxt).start()
    for g in range(max(0, n - _NBUF), n):
        st(g).wait()


def kernel(inputs_embeds, embed_table):
    del embed_table
    b, s, h = inputs_embeds.shape
    x = inputs_embeds.reshape(b * s, h)
    rows = b * s
    out = pl.pallas_call(
        _copy_body,
        in_specs=[pl.BlockSpec(memory_space=pl.ANY)],
        out_specs=pl.BlockSpec(memory_space=pl.ANY),
        out_shape=jax.ShapeDtypeStruct((rows, h), x.dtype),
        scratch_shapes=[
            pltpu.VMEM((_NBUF, _CH, h), jnp.float32),
            pltpu.SemaphoreType.DMA((_NBUF,)),
            pltpu.SemaphoreType.DMA((_NBUF,)),
        ],
    )(x)
    return out.reshape(b, s, h)


# speedup vs baseline: 1.0137x; 1.0137x over previous
import jax
import jax.numpy as jnp
from jax.experimental import pallas as pl
from jax.experimental.pallas import tpu as pltpu

_CH = 512
_NBUF = 8


def _schedule(rows):
    ramp = [64, 64, 128, 256]
    tail = [256, 128, 64, 64]
    body = rows - sum(ramp) - sum(tail)
    sizes = ramp + [_CH] * (body // _CH) + tail
    chunks, off = [], 0
    for sz in sizes:
        chunks.append((off, sz))
        off += sz
    assert off == rows
    return chunks


def _copy_body(x_ref, o_ref, buf, ld_sems, st_sems):
    chunks = _schedule(x_ref.shape[0])
    n = len(chunks)

    def ld(g):
        slot = g % _NBUF
        off, sz = chunks[g]
        return pltpu.make_async_copy(
            x_ref.at[pl.ds(off, sz)], buf.at[slot, pl.ds(0, sz)],
            ld_sems.at[slot])

    def st(g):
        slot = g % _NBUF
        off, sz = chunks[g]
        return pltpu.make_async_copy(
            buf.at[slot, pl.ds(0, sz)], o_ref.at[pl.ds(off, sz)],
            st_sems.at[slot])

    k = _NBUF // 2
    for g in range(min(k, n)):
        ld(g).start()
    for g in range(n):
        ld(g).wait()
        st(g).start()
        nxt = g + k
        if nxt < n:
            prev = nxt - _NBUF
            if prev >= 0:
                st(prev).wait()
            ld(nxt).start()
    for g in range(max(0, n - _NBUF), n):
        st(g).wait()


def kernel(inputs_embeds, embed_table):
    del embed_table
    b, s, h = inputs_embeds.shape
    x = inputs_embeds.reshape(b * s, h)
    rows = b * s
    out = pl.pallas_call(
        _copy_body,
        in_specs=[pl.BlockSpec(memory_space=pl.ANY)],
        out_specs=pl.BlockSpec(memory_space=pl.ANY),
        out_shape=jax.ShapeDtypeStruct((rows, h), x.dtype),
        scratch_shapes=[
            pltpu.VMEM((_NBUF, _CH, h), jnp.float32),
            pltpu.SemaphoreType.DMA((_NBUF,)),
            pltpu.SemaphoreType.DMA((_NBUF,)),
        ],
    )(x)
    return out.reshape(b, s, h)
